# baseline (speedup 1.0000x reference)
import jax
import jax.numpy as jnp
from jax import lax
from jax.experimental import pallas as pl
from jax.experimental.pallas import tpu as pltpu
from jax.experimental.pallas import tpu_sc as plsc

GRID_H = 256
GRID_W = 256
N_CODES = 1024
F_DIM = 8
B_PTS = 65536

NUM_SC_CORES = 2
NUM_SUBCORES = 16
LANES = 16
NW = NUM_SC_CORES * NUM_SUBCORES
BPW = B_PTS // NW
GROUPS = BPW // LANES


_A_ROWS = 4096


def _argmax_body(hm_ref, out_ref):
    v = hm_ref[...]
    m = jnp.max(v, axis=1, keepdims=True)
    ii = lax.broadcasted_iota(jnp.int32, v.shape, 1)
    sel = jnp.where(v == m, ii, N_CODES)
    idx = jnp.min(sel, axis=1) << 3
    out_ref[...] = idx.reshape(_A_ROWS // 128, 128)


def _stage_a(hm2):
    n_rows = hm2.shape[0]
    return pl.pallas_call(
        _argmax_body,
        grid=(n_rows // _A_ROWS,),
        in_specs=[pl.BlockSpec((_A_ROWS, N_CODES), lambda i: (i, 0))],
        out_specs=pl.BlockSpec((_A_ROWS // 128, 128), lambda i: (i, 0)),
        out_shape=jax.ShapeDtypeStruct((n_rows // 128, 128), jnp.int32),
    )(hm2)


def _sc_body(xq_hbm, yq_hbm, idx_hbm, hf_hbm, out_hbm,
             idxmap_v, hf_v, xq_v, yq_v, feats_v):
    c = lax.axis_index("c")
    s = lax.axis_index("s")
    wid = s * NUM_SC_CORES + c
    base = wid * BPW
    pltpu.sync_copy(idx_hbm, idxmap_v)
    pltpu.sync_copy(hf_hbm, hf_v)
    pltpu.sync_copy(xq_hbm.at[pl.ds(base, BPW)], xq_v)
    pltpu.sync_copy(yq_hbm.at[pl.ds(base, BPW)], yq_v)

    lane = lax.iota(jnp.int32, LANES)

    @plsc.parallel_loop(0, GROUPS, unroll=4)
    def group(g):
        q0 = g * LANES
        xv = xq_v[pl.ds(q0, LANES)]
        yv = yq_v[pl.ds(q0, LANES)]
        xs = xv * float(GRID_H)
        ys = yv * float(GRID_W)
        xi = xs.astype(jnp.int32)
        yi = ys.astype(jnp.int32)
        wx = xs - xi.astype(jnp.float32)
        wy = ys - yi.astype(jnp.float32)
        xi1 = jnp.minimum(xi + 1, GRID_H - 1)
        yi1 = jnp.minimum(yi + 1, GRID_W - 1)
        r0 = xi << 8
        r1 = xi1 << 8
        c00 = plsc.load_gather(idxmap_v, [r0 + yi])
        c01 = plsc.load_gather(idxmap_v, [r0 + yi1])
        c10 = plsc.load_gather(idxmap_v, [r1 + yi])
        c11 = plsc.load_gather(idxmap_v, [r1 + yi1])
        omx = 1.0 - wx
        omy = 1.0 - wy
        w00 = omx * omy
        w01 = omx * wy
        w10 = wx * omy
        w11 = wx * wy
        qloc8 = (q0 + lane) << 3
        for f in range(F_DIM):
            a00 = plsc.load_gather(hf_v, [c00 + f])
            a01 = plsc.load_gather(hf_v, [c01 + f])
            a10 = plsc.load_gather(hf_v, [c10 + f])
            a11 = plsc.load_gather(hf_v, [c11 + f])
            acc = w00 * a00 + w01 * a01 + w10 * a10 + w11 * a11
            plsc.store_scatter(feats_v, [qloc8 + f], acc)
    pltpu.sync_copy(feats_v, out_hbm.at[pl.ds(base * F_DIM, BPW * F_DIM)])


_SC_CALL_CACHE = []


def _sc_call(*args):
    if not _SC_CALL_CACHE:
        _SC_CALL_CACHE.append(pl.kernel(
            _sc_body,
            out_type=jax.ShapeDtypeStruct((B_PTS * F_DIM,), jnp.float32),
            mesh=plsc.VectorSubcoreMesh(
                core_axis_name="c", subcore_axis_name="s",
                num_cores=NUM_SC_CORES, num_subcores=NUM_SUBCORES),
            compiler_params=pltpu.CompilerParams(needs_layout_passes=False),
            scratch_types=[
                pltpu.VMEM((GRID_H * GRID_W,), jnp.int32),
                pltpu.VMEM((N_CODES * F_DIM,), jnp.float32),
                pltpu.VMEM((BPW,), jnp.float32),
                pltpu.VMEM((BPW,), jnp.float32),
                pltpu.VMEM((BPW * F_DIM,), jnp.float32),
            ],
        ))
    return _SC_CALL_CACHE[0](*args)


_C_PACK = 128 // F_DIM
_C_ROWS = 1024


def _mlp_body(f_ref, w1_ref, w2_ref, o_ref):
    f = f_ref[...]
    h = jnp.maximum(
        lax.dot(f, w1_ref[...], preferred_element_type=jnp.float32), 0.0)
    z = lax.dot(h, w2_ref[...], preferred_element_type=jnp.float32)
    o_ref[...] = 1.0 / (1.0 + jnp.exp(-z))


def _stage_c(feats2, W1b, W2b):
    n_rows = B_PTS // _C_PACK
    return pl.pallas_call(
        _mlp_body,
        grid=(n_rows // _C_ROWS,),
        in_specs=[
            pl.BlockSpec((_C_ROWS, 128), lambda i: (i, 0)),
            pl.BlockSpec((128, 32 * _C_PACK), lambda i: (0, 0)),
            pl.BlockSpec((32 * _C_PACK, 3 * _C_PACK), lambda i: (0, 0)),
        ],
        out_specs=pl.BlockSpec((_C_ROWS, 3 * _C_PACK), lambda i: (i, 0)),
        out_shape=jax.ShapeDtypeStruct((n_rows, 3 * _C_PACK), jnp.float32),
    )(feats2, W1b, W2b)


def kernel(x, hashmap, hash_features, W1, W2):
    hm2 = hashmap.reshape(GRID_H * GRID_W, N_CODES)
    idx_map = _stage_a(hm2).reshape(GRID_H * GRID_W)
    xq = x[:, 0]
    yq = x[:, 1]
    hf_flat = hash_features.reshape(N_CODES * F_DIM)
    feats2 = _sc_call(xq, yq, idx_map, hf_flat).reshape(B_PTS // _C_PACK, 128)
    eye = jnp.eye(_C_PACK, dtype=jnp.float32)
    W1b = jnp.kron(eye, W1)
    W2b = jnp.kron(eye, W2)
    out = _stage_c(feats2, W1b, W2b)
    p = out.reshape(B_PTS // _C_PACK, _C_PACK, 3)
    return jnp.stack(
        [p[:, :, 0].reshape(B_PTS), p[:, :, 1].reshape(B_PTS),
         p[:, :, 2].reshape(B_PTS)], axis=1)


# speedup vs baseline: 1.0117x; 1.0117x over previous
import jax
import jax.numpy as jnp
from jax import lax
from jax.experimental import pallas as pl
from jax.experimental.pallas import tpu as pltpu
from jax.experimental.pallas import tpu_sc as plsc

GRID_H = 256
GRID_W = 256
N_CODES = 1024
F_DIM = 8
B_PTS = 65536

NUM_SC_CORES = 2
NUM_SUBCORES = 16
LANES = 16
NW = NUM_SC_CORES * NUM_SUBCORES
BPW = B_PTS // NW
GROUPS = BPW // LANES


_A_ROWS = 4096


def _argmax_body(hm_ref, out_ref):
    v = hm_ref[...]
    m = jnp.max(v, axis=1, keepdims=True)
    ii = lax.broadcasted_iota(jnp.int32, v.shape, 1)
    sel = jnp.where(v == m, ii, N_CODES)
    idx = jnp.min(sel, axis=1) << 3
    out_ref[...] = idx.reshape(_A_ROWS // 128, 128)


def _stage_a(hm2):
    n_rows = hm2.shape[0]
    return pl.pallas_call(
        _argmax_body,
        grid=(n_rows // _A_ROWS,),
        in_specs=[pl.BlockSpec((_A_ROWS, N_CODES), lambda i: (i, 0))],
        out_specs=pl.BlockSpec((_A_ROWS // 128, 128), lambda i: (i, 0)),
        out_shape=jax.ShapeDtypeStruct((n_rows // 128, 128), jnp.int32),
    )(hm2)


def _sc_body(xq_hbm, yq_hbm, idx_hbm, hf_hbm, out_hbm,
             idxmap_v, hf_v, xq_v, yq_v, feats_v):
    c = lax.axis_index("c")
    s = lax.axis_index("s")
    wid = s * NUM_SC_CORES + c
    base = wid * BPW
    pltpu.sync_copy(idx_hbm, idxmap_v)
    pltpu.sync_copy(hf_hbm, hf_v)
    pltpu.sync_copy(xq_hbm.at[pl.ds(base, BPW)], xq_v)
    pltpu.sync_copy(yq_hbm.at[pl.ds(base, BPW)], yq_v)

    lane = lax.iota(jnp.int32, LANES)

    @plsc.parallel_loop(0, GROUPS, unroll=2)
    def group(g):
        q0 = g * LANES
        xv = xq_v[pl.ds(q0, LANES)]
        yv = yq_v[pl.ds(q0, LANES)]
        xs = xv * float(GRID_H)
        ys = yv * float(GRID_W)
        xi = xs.astype(jnp.int32)
        yi = ys.astype(jnp.int32)
        wx = xs - xi.astype(jnp.float32)
        wy = ys - yi.astype(jnp.float32)
        xi1 = jnp.minimum(xi + 1, GRID_H - 1)
        yi1 = jnp.minimum(yi + 1, GRID_W - 1)
        r0 = xi << 8
        r1 = xi1 << 8
        c00 = plsc.load_gather(idxmap_v, [r0 + yi])
        c01 = plsc.load_gather(idxmap_v, [r0 + yi1])
        c10 = plsc.load_gather(idxmap_v, [r1 + yi])
        c11 = plsc.load_gather(idxmap_v, [r1 + yi1])
        omx = 1.0 - wx
        omy = 1.0 - wy
        w00 = omx * omy
        w01 = omx * wy
        w10 = wx * omy
        w11 = wx * wy
        qloc8 = (q0 + lane) << 3
        for f in range(F_DIM):
            a00 = plsc.load_gather(hf_v, [c00 + f])
            a01 = plsc.load_gather(hf_v, [c01 + f])
            a10 = plsc.load_gather(hf_v, [c10 + f])
            a11 = plsc.load_gather(hf_v, [c11 + f])
            acc = w00 * a00 + w01 * a01 + w10 * a10 + w11 * a11
            plsc.store_scatter(feats_v, [qloc8 + f], acc)
    pltpu.sync_copy(feats_v, out_hbm.at[pl.ds(base * F_DIM, BPW * F_DIM)])


_SC_CALL_CACHE = []


def _sc_call(*args):
    if not _SC_CALL_CACHE:
        _SC_CALL_CACHE.append(pl.kernel(
            _sc_body,
            out_type=jax.ShapeDtypeStruct((B_PTS * F_DIM,), jnp.float32),
            mesh=plsc.VectorSubcoreMesh(
                core_axis_name="c", subcore_axis_name="s",
                num_cores=NUM_SC_CORES, num_subcores=NUM_SUBCORES),
            compiler_params=pltpu.CompilerParams(needs_layout_passes=False),
            scratch_types=[
                pltpu.VMEM((GRID_H * GRID_W,), jnp.int32),
                pltpu.VMEM((N_CODES * F_DIM,), jnp.float32),
                pltpu.VMEM((BPW,), jnp.float32),
                pltpu.VMEM((BPW,), jnp.float32),
                pltpu.VMEM((BPW * F_DIM,), jnp.float32),
            ],
        ))
    return _SC_CALL_CACHE[0](*args)


_C_PACK = 128 // F_DIM
_C_ROWS = 1024


def _mlp_body(f_ref, w1_ref, w2_ref, o_ref):
    f = f_ref[...]
    h = jnp.maximum(
        lax.dot(f, w1_ref[...], preferred_element_type=jnp.float32), 0.0)
    z = lax.dot(h, w2_ref[...], preferred_element_type=jnp.float32)
    o_ref[...] = 1.0 / (1.0 + jnp.exp(-z))


def _stage_c(feats2, W1b, W2b):
    n_rows = B_PTS // _C_PACK
    return pl.pallas_call(
        _mlp_body,
        grid=(n_rows // _C_ROWS,),
        in_specs=[
            pl.BlockSpec((_C_ROWS, 128), lambda i: (i, 0)),
            pl.BlockSpec((128, 32 * _C_PACK), lambda i: (0, 0)),
            pl.BlockSpec((32 * _C_PACK, 3 * _C_PACK), lambda i: (0, 0)),
        ],
        out_specs=pl.BlockSpec((_C_ROWS, 3 * _C_PACK), lambda i: (i, 0)),
        out_shape=jax.ShapeDtypeStruct((n_rows, 3 * _C_PACK), jnp.float32),
    )(feats2, W1b, W2b)


def kernel(x, hashmap, hash_features, W1, W2):
    hm2 = hashmap.reshape(GRID_H * GRID_W, N_CODES)
    idx_map = _stage_a(hm2).reshape(GRID_H * GRID_W)
    xq = x[:, 0]
    yq = x[:, 1]
    hf_flat = hash_features.reshape(N_CODES * F_DIM)
    feats2 = _sc_call(xq, yq, idx_map, hf_flat).reshape(B_PTS // _C_PACK, 128)
    eye = jnp.eye(_C_PACK, dtype=jnp.float32)
    W1b = jnp.kron(eye, W1)
    W2b = jnp.kron(eye, W2)
    out = _stage_c(feats2, W1b, W2b)
    p = out.reshape(B_PTS // _C_PACK, _C_PACK, 3)
    return jnp.stack(
        [p[:, :, 0].reshape(B_PTS), p[:, :, 1].reshape(B_PTS),
         p[:, :, 2].reshape(B_PTS)], axis=1)


# speedup vs baseline: 1.0198x; 1.0080x over previous
import jax
import jax.numpy as jnp
from jax import lax
from jax.experimental import pallas as pl
from jax.experimental.pallas import tpu as pltpu
from jax.experimental.pallas import tpu_sc as plsc

GRID_H = 256
GRID_W = 256
N_CODES = 1024
F_DIM = 8
B_PTS = 65536

NUM_SC_CORES = 2
NUM_SUBCORES = 16
LANES = 16
NW = NUM_SC_CORES * NUM_SUBCORES
BPW = B_PTS // NW
GROUPS = BPW // LANES


_A_ROWS = 4096


def _argmax_body(hm_ref, out_ref):
    v = hm_ref[...]
    m = jnp.max(v, axis=1, keepdims=True)
    ii = lax.broadcasted_iota(jnp.int32, v.shape, 1)
    sel = jnp.where(v == m, ii, N_CODES)
    idx = jnp.min(sel, axis=1) << 3
    out_ref[...] = idx.reshape(_A_ROWS // 128, 128)


def _stage_a(hm2):
    n_rows = hm2.shape[0]
    return pl.pallas_call(
        _argmax_body,
        grid=(n_rows // _A_ROWS,),
        in_specs=[pl.BlockSpec((_A_ROWS, N_CODES), lambda i: (i, 0))],
        out_specs=pl.BlockSpec((_A_ROWS // 128, 128), lambda i: (i, 0)),
        out_shape=jax.ShapeDtypeStruct((n_rows // 128, 128), jnp.int32),
    )(hm2)


def _sc_body(xq_hbm, yq_hbm, idx_hbm, hf_hbm, out_hbm,
             idxmap_v, hf_v, xq_v, yq_v, feats_v):
    c = lax.axis_index("c")
    s = lax.axis_index("s")
    wid = s * NUM_SC_CORES + c
    base = wid * BPW
    pltpu.sync_copy(idx_hbm, idxmap_v)
    pltpu.sync_copy(hf_hbm, hf_v)
    pltpu.sync_copy(xq_hbm.at[pl.ds(base, BPW)], xq_v)
    pltpu.sync_copy(yq_hbm.at[pl.ds(base, BPW)], yq_v)

    lane = lax.iota(jnp.int32, LANES)

    def group(g, carry):
        q0 = g * LANES
        xv = xq_v[pl.ds(q0, LANES)]
        yv = yq_v[pl.ds(q0, LANES)]
        xs = xv * float(GRID_H)
        ys = yv * float(GRID_W)
        xi = xs.astype(jnp.int32)
        yi = ys.astype(jnp.int32)
        wx = xs - xi.astype(jnp.float32)
        wy = ys - yi.astype(jnp.float32)
        xi1 = jnp.minimum(xi + 1, GRID_H - 1)
        yi1 = jnp.minimum(yi + 1, GRID_W - 1)
        r0 = xi << 8
        r1 = xi1 << 8
        c00 = plsc.load_gather(idxmap_v, [r0 + yi])
        c01 = plsc.load_gather(idxmap_v, [r0 + yi1])
        c10 = plsc.load_gather(idxmap_v, [r1 + yi])
        c11 = plsc.load_gather(idxmap_v, [r1 + yi1])
        omx = 1.0 - wx
        omy = 1.0 - wy
        w00 = omx * omy
        w01 = omx * wy
        w10 = wx * omy
        w11 = wx * wy
        qloc8 = (q0 + lane) << 3
        for f in range(F_DIM):
            a00 = plsc.load_gather(hf_v, [c00 + f])
            a01 = plsc.load_gather(hf_v, [c01 + f])
            a10 = plsc.load_gather(hf_v, [c10 + f])
            a11 = plsc.load_gather(hf_v, [c11 + f])
            acc = w00 * a00 + w01 * a01 + w10 * a10 + w11 * a11
            plsc.store_scatter(feats_v, [qloc8 + f], acc)
        return carry

    lax.fori_loop(0, GROUPS, group, 0)
    pltpu.sync_copy(feats_v, out_hbm.at[pl.ds(base * F_DIM, BPW * F_DIM)])


_SC_CALL_CACHE = []


def _sc_call(*args):
    if not _SC_CALL_CACHE:
        _SC_CALL_CACHE.append(pl.kernel(
            _sc_body,
            out_type=jax.ShapeDtypeStruct((B_PTS * F_DIM,), jnp.float32),
            mesh=plsc.VectorSubcoreMesh(
                core_axis_name="c", subcore_axis_name="s",
                num_cores=NUM_SC_CORES, num_subcores=NUM_SUBCORES),
            compiler_params=pltpu.CompilerParams(needs_layout_passes=False),
            scratch_types=[
                pltpu.VMEM((GRID_H * GRID_W,), jnp.int32),
                pltpu.VMEM((N_CODES * F_DIM,), jnp.float32),
                pltpu.VMEM((BPW,), jnp.float32),
                pltpu.VMEM((BPW,), jnp.float32),
                pltpu.VMEM((BPW * F_DIM,), jnp.float32),
            ],
        ))
    return _SC_CALL_CACHE[0](*args)


_C_PACK = 128 // F_DIM
_C_ROWS = 1024


def _mlp_body(f_ref, w1_ref, w2_ref, o_ref):
    f = f_ref[...]
    h = jnp.maximum(
        lax.dot(f, w1_ref[...], preferred_element_type=jnp.float32), 0.0)
    z = lax.dot(h, w2_ref[...], preferred_element_type=jnp.float32)
    o_ref[...] = 1.0 / (1.0 + jnp.exp(-z))


def _stage_c(feats2, W1b, W2b):
    n_rows = B_PTS // _C_PACK
    return pl.pallas_call(
        _mlp_body,
        grid=(n_rows // _C_ROWS,),
        in_specs=[
            pl.BlockSpec((_C_ROWS, 128), lambda i: (i, 0)),
            pl.BlockSpec((128, 32 * _C_PACK), lambda i: (0, 0)),
            pl.BlockSpec((32 * _C_PACK, 3 * _C_PACK), lambda i: (0, 0)),
        ],
        out_specs=pl.BlockSpec((_C_ROWS, 3 * _C_PACK), lambda i: (i, 0)),
        out_shape=jax.ShapeDtypeStruct((n_rows, 3 * _C_PACK), jnp.float32),
    )(feats2, W1b, W2b)


def kernel(x, hashmap, hash_features, W1, W2):
    hm2 = hashmap.reshape(GRID_H * GRID_W, N_CODES)
    idx_map = _stage_a(hm2).reshape(GRID_H * GRID_W)
    xq = x[:, 0]
    yq = x[:, 1]
    hf_flat = hash_features.reshape(N_CODES * F_DIM)
    feats2 = _sc_call(xq, yq, idx_map, hf_flat).reshape(B_PTS // _C_PACK, 128)
    eye = jnp.eye(_C_PACK, dtype=jnp.float32)
    W1b = jnp.kron(eye, W1)
    W2b = jnp.kron(eye, W2)
    out = _stage_c(feats2, W1b, W2b)
    p = out.reshape(B_PTS // _C_PACK, _C_PACK, 3)
    return jnp.stack(
        [p[:, :, 0].reshape(B_PTS), p[:, :, 1].reshape(B_PTS),
         p[:, :, 2].reshape(B_PTS)], axis=1)


# speedup vs baseline: 1.0558x; 1.0353x over previous
import jax
import jax.numpy as jnp
from jax import lax
from jax.experimental import pallas as pl
from jax.experimental.pallas import tpu as pltpu
from jax.experimental.pallas import tpu_sc as plsc

GRID_H = 256
GRID_W = 256
N_CODES = 1024
F_DIM = 8
B_PTS = 65536

NUM_SC_CORES = 2
NUM_SUBCORES = 16
LANES = 16
NW = NUM_SC_CORES * NUM_SUBCORES
BPW = B_PTS // NW
GROUPS = BPW // LANES


_A_ROWS = 4096


def _argmax_body(hm_ref, out_ref):
    v = hm_ref[...]
    m = jnp.max(v, axis=1, keepdims=True)
    ii = lax.broadcasted_iota(jnp.int32, v.shape, 1)
    sel = jnp.where(v == m, ii, N_CODES)
    idx = jnp.min(sel, axis=1) << 3
    out_ref[...] = idx.reshape(_A_ROWS // 128, 128)


def _stage_a(hm2):
    n_rows = hm2.shape[0]
    return pl.pallas_call(
        _argmax_body,
        grid=(n_rows // _A_ROWS,),
        in_specs=[pl.BlockSpec((_A_ROWS, N_CODES), lambda i: (i, 0))],
        out_specs=pl.BlockSpec((_A_ROWS // 128, 128), lambda i: (i, 0)),
        out_shape=jax.ShapeDtypeStruct((n_rows // 128, 128), jnp.int32),
    )(hm2)


def _sc_body(xq_hbm, yq_hbm, idx_hbm, hf_hbm, out_hbm,
             idxmap_v, hf_v, xq_v, yq_v, feats_v, idx_sh):
    c = lax.axis_index("c")
    s = lax.axis_index("s")
    wid = s * NUM_SC_CORES + c
    base = wid * BPW
    @pl.when(s == 0)
    def _stage_idx():
        pltpu.sync_copy(idx_hbm, idx_sh)

    pltpu.sync_copy(hf_hbm, hf_v)
    pltpu.sync_copy(xq_hbm.at[pl.ds(base, BPW)], xq_v)
    pltpu.sync_copy(yq_hbm.at[pl.ds(base, BPW)], yq_v)
    plsc.subcore_barrier()
    pltpu.sync_copy(idx_sh, idxmap_v)

    lane = lax.iota(jnp.int32, LANES)

    def group(g, carry):
        q0 = g * LANES
        xv = xq_v[pl.ds(q0, LANES)]
        yv = yq_v[pl.ds(q0, LANES)]
        xs = xv * float(GRID_H)
        ys = yv * float(GRID_W)
        xi = xs.astype(jnp.int32)
        yi = ys.astype(jnp.int32)
        wx = xs - xi.astype(jnp.float32)
        wy = ys - yi.astype(jnp.float32)
        xi1 = jnp.minimum(xi + 1, GRID_H - 1)
        yi1 = jnp.minimum(yi + 1, GRID_W - 1)
        r0 = xi << 8
        r1 = xi1 << 8
        c00 = plsc.load_gather(idxmap_v, [r0 + yi])
        c01 = plsc.load_gather(idxmap_v, [r0 + yi1])
        c10 = plsc.load_gather(idxmap_v, [r1 + yi])
        c11 = plsc.load_gather(idxmap_v, [r1 + yi1])
        omx = 1.0 - wx
        omy = 1.0 - wy
        w00 = omx * omy
        w01 = omx * wy
        w10 = wx * omy
        w11 = wx * wy
        qloc8 = (q0 + lane) << 3
        for f in range(F_DIM):
            a00 = plsc.load_gather(hf_v, [c00 + f])
            a01 = plsc.load_gather(hf_v, [c01 + f])
            a10 = plsc.load_gather(hf_v, [c10 + f])
            a11 = plsc.load_gather(hf_v, [c11 + f])
            acc = w00 * a00 + w01 * a01 + w10 * a10 + w11 * a11
            plsc.store_scatter(feats_v, [qloc8 + f], acc)
        return carry

    lax.fori_loop(0, GROUPS, group, 0)
    pltpu.sync_copy(feats_v, out_hbm.at[pl.ds(base * F_DIM, BPW * F_DIM)])


_SC_CALL_CACHE = []


def _sc_call(*args):
    if not _SC_CALL_CACHE:
        _SC_CALL_CACHE.append(pl.kernel(
            _sc_body,
            out_type=jax.ShapeDtypeStruct((B_PTS * F_DIM,), jnp.float32),
            mesh=plsc.VectorSubcoreMesh(
                core_axis_name="c", subcore_axis_name="s",
                num_cores=NUM_SC_CORES, num_subcores=NUM_SUBCORES),
            compiler_params=pltpu.CompilerParams(needs_layout_passes=False),
            scratch_types=[
                pltpu.VMEM((GRID_H * GRID_W,), jnp.int32),
                pltpu.VMEM((N_CODES * F_DIM,), jnp.float32),
                pltpu.VMEM((BPW,), jnp.float32),
                pltpu.VMEM((BPW,), jnp.float32),
                pltpu.VMEM((BPW * F_DIM,), jnp.float32),
                pltpu.VMEM_SHARED((GRID_H * GRID_W,), jnp.int32),
            ],
        ))
    return _SC_CALL_CACHE[0](*args)


_C_PACK = 128 // F_DIM
_C_ROWS = 1024


def _mlp_body(f_ref, w1_ref, w2_ref, o_ref):
    f = f_ref[...]
    h = jnp.maximum(
        lax.dot(f, w1_ref[...], preferred_element_type=jnp.float32), 0.0)
    z = lax.dot(h, w2_ref[...], preferred_element_type=jnp.float32)
    o_ref[...] = 1.0 / (1.0 + jnp.exp(-z))


def _stage_c(feats2, W1b, W2b):
    n_rows = B_PTS // _C_PACK
    return pl.pallas_call(
        _mlp_body,
        grid=(n_rows // _C_ROWS,),
        in_specs=[
            pl.BlockSpec((_C_ROWS, 128), lambda i: (i, 0)),
            pl.BlockSpec((128, 32 * _C_PACK), lambda i: (0, 0)),
            pl.BlockSpec((32 * _C_PACK, 3 * _C_PACK), lambda i: (0, 0)),
        ],
        out_specs=pl.BlockSpec((_C_ROWS, 3 * _C_PACK), lambda i: (i, 0)),
        out_shape=jax.ShapeDtypeStruct((n_rows, 3 * _C_PACK), jnp.float32),
    )(feats2, W1b, W2b)


def kernel(x, hashmap, hash_features, W1, W2):
    hm2 = hashmap.reshape(GRID_H * GRID_W, N_CODES)
    idx_map = _stage_a(hm2).reshape(GRID_H * GRID_W)
    xq = x[:, 0]
    yq = x[:, 1]
    hf_flat = hash_features.reshape(N_CODES * F_DIM)
    feats2 = _sc_call(xq, yq, idx_map, hf_flat).reshape(B_PTS // _C_PACK, 128)
    eye = jnp.eye(_C_PACK, dtype=jnp.float32)
    W1b = jnp.kron(eye, W1)
    W2b = jnp.kron(eye, W2)
    out = _stage_c(feats2, W1b, W2b)
    p = out.reshape(B_PTS // _C_PACK, _C_PACK, 3)
    return jnp.stack(
        [p[:, :, 0].reshape(B_PTS), p[:, :, 1].reshape(B_PTS),
         p[:, :, 2].reshape(B_PTS)], axis=1)


# speedup vs baseline: 1.0565x; 1.0006x over previous
import jax
import jax.numpy as jnp
from jax import lax
from jax.experimental import pallas as pl
from jax.experimental.pallas import tpu as pltpu
from jax.experimental.pallas import tpu_sc as plsc

GRID_H = 256
GRID_W = 256
N_CODES = 1024
F_DIM = 8
B_PTS = 65536

NUM_SC_CORES = 2
NUM_SUBCORES = 16
LANES = 16
NW = NUM_SC_CORES * NUM_SUBCORES
BPW = B_PTS // NW
GROUPS = BPW // LANES


_A_ROWS = 4096


def _argmax_body(hm_ref, out_ref):
    v = hm_ref[...]
    m = jnp.max(v, axis=1, keepdims=True)
    ii = lax.broadcasted_iota(jnp.int32, v.shape, 1)
    sel = jnp.where(v == m, ii, N_CODES)
    idx = jnp.min(sel, axis=1) << 3
    out_ref[...] = idx.reshape(_A_ROWS // 128, 128)


def _stage_a(hm2):
    n_rows = hm2.shape[0]
    return pl.pallas_call(
        _argmax_body,
        grid=(n_rows // _A_ROWS,),
        in_specs=[pl.BlockSpec((_A_ROWS, N_CODES), lambda i: (i, 0))],
        out_specs=pl.BlockSpec((_A_ROWS // 128, 128), lambda i: (i, 0)),
        out_shape=jax.ShapeDtypeStruct((n_rows // 128, 128), jnp.int32),
    )(hm2)


def _sc_body(xq_hbm, yq_hbm, idx_hbm, hf_hbm, out_hbm,
             idxmap_v, hf_v, xq_v, yq_v, feats_v, idx_sh):
    c = lax.axis_index("c")
    s = lax.axis_index("s")
    wid = s * NUM_SC_CORES + c
    base = wid * BPW
    @pl.when(s == 0)
    def _stage_idx():
        pltpu.sync_copy(idx_hbm, idx_sh)

    pltpu.sync_copy(hf_hbm, hf_v)
    pltpu.sync_copy(xq_hbm.at[pl.ds(base, BPW)], xq_v)
    pltpu.sync_copy(yq_hbm.at[pl.ds(base, BPW)], yq_v)
    plsc.subcore_barrier()
    pltpu.sync_copy(idx_sh, idxmap_v)

    lane = lax.iota(jnp.int32, LANES)

    def group(g, carry):
        q0 = g * LANES
        xv = xq_v[pl.ds(q0, LANES)]
        yv = yq_v[pl.ds(q0, LANES)]
        xs = xv * float(GRID_H)
        ys = yv * float(GRID_W)
        xi = xs.astype(jnp.int32)
        yi = ys.astype(jnp.int32)
        wx = xs - xi.astype(jnp.float32)
        wy = ys - yi.astype(jnp.float32)
        xi1 = jnp.minimum(xi + 1, GRID_H - 1)
        yi1 = jnp.minimum(yi + 1, GRID_W - 1)
        r0 = xi << 8
        r1 = xi1 << 8
        c00 = plsc.load_gather(idxmap_v, [r0 + yi])
        c01 = plsc.load_gather(idxmap_v, [r0 + yi1])
        c10 = plsc.load_gather(idxmap_v, [r1 + yi])
        c11 = plsc.load_gather(idxmap_v, [r1 + yi1])
        omx = 1.0 - wx
        omy = 1.0 - wy
        w00 = omx * omy
        w01 = omx * wy
        w10 = wx * omy
        w11 = wx * wy
        qloc8 = (q0 + lane) << 3
        for f in range(F_DIM):
            a00 = plsc.load_gather(hf_v, [c00 + f])
            a01 = plsc.load_gather(hf_v, [c01 + f])
            a10 = plsc.load_gather(hf_v, [c10 + f])
            a11 = plsc.load_gather(hf_v, [c11 + f])
            acc = w00 * a00 + w01 * a01 + w10 * a10 + w11 * a11
            plsc.store_scatter(feats_v, [qloc8 + f], acc)
        return carry

    lax.fori_loop(0, GROUPS, group, 0)
    pltpu.sync_copy(feats_v, out_hbm.at[pl.ds(base * F_DIM, BPW * F_DIM)])


_SC_CALL_CACHE = []


def _sc_call(*args):
    if not _SC_CALL_CACHE:
        _SC_CALL_CACHE.append(pl.kernel(
            _sc_body,
            out_type=jax.ShapeDtypeStruct((B_PTS * F_DIM,), jnp.float32),
            mesh=plsc.VectorSubcoreMesh(
                core_axis_name="c", subcore_axis_name="s",
                num_cores=NUM_SC_CORES, num_subcores=NUM_SUBCORES),
            compiler_params=pltpu.CompilerParams(needs_layout_passes=False),
            scratch_types=[
                pltpu.VMEM((GRID_H * GRID_W,), jnp.int32),
                pltpu.VMEM((N_CODES * F_DIM,), jnp.float32),
                pltpu.VMEM((BPW,), jnp.float32),
                pltpu.VMEM((BPW,), jnp.float32),
                pltpu.VMEM((BPW * F_DIM,), jnp.float32),
                pltpu.VMEM_SHARED((GRID_H * GRID_W,), jnp.int32),
            ],
        ))
    return _SC_CALL_CACHE[0](*args)


_C_PACK = 128 // F_DIM
_C_ROWS = 1024


def _mlp_body(f_ref, w1_ref, w2_ref, o_ref, w1b_ref, w2b_ref):
    @pl.when(pl.program_id(0) == 0)
    def _build():
        w1b_ref[...] = jnp.zeros((128, 32 * _C_PACK), jnp.float32)
        w2b_ref[...] = jnp.zeros((32 * _C_PACK, 3 * _C_PACK), jnp.float32)
        w1 = w1_ref[...]
        w2 = w2_ref[...]
        for q in range(_C_PACK):
            w1b_ref[q * F_DIM:(q + 1) * F_DIM, q * 32:(q + 1) * 32] = w1
            w2b_ref[q * 32:(q + 1) * 32, q * 3:(q + 1) * 3] = w2

    f = f_ref[...]
    h = jnp.maximum(
        lax.dot(f, w1b_ref[...], preferred_element_type=jnp.float32), 0.0)
    z = lax.dot(h, w2b_ref[...], preferred_element_type=jnp.float32)
    o_ref[...] = 1.0 / (1.0 + jnp.exp(-z))


def _stage_c(feats2, W1, W2):
    n_rows = B_PTS // _C_PACK
    return pl.pallas_call(
        _mlp_body,
        grid=(n_rows // _C_ROWS,),
        in_specs=[
            pl.BlockSpec((_C_ROWS, 128), lambda i: (i, 0)),
            pl.BlockSpec((F_DIM, 32), lambda i: (0, 0)),
            pl.BlockSpec((32, 3), lambda i: (0, 0)),
        ],
        out_specs=pl.BlockSpec((_C_ROWS, 3 * _C_PACK), lambda i: (i, 0)),
        out_shape=jax.ShapeDtypeStruct((n_rows, 3 * _C_PACK), jnp.float32),
        scratch_shapes=[
            pltpu.VMEM((128, 32 * _C_PACK), jnp.float32),
            pltpu.VMEM((32 * _C_PACK, 3 * _C_PACK), jnp.float32),
        ],
    )(feats2, W1, W2)


def kernel(x, hashmap, hash_features, W1, W2):
    hm2 = hashmap.reshape(GRID_H * GRID_W, N_CODES)
    idx_map = _stage_a(hm2).reshape(GRID_H * GRID_W)
    xq = x[:, 0]
    yq = x[:, 1]
    hf_flat = hash_features.reshape(N_CODES * F_DIM)
    feats2 = _sc_call(xq, yq, idx_map, hf_flat).reshape(B_PTS // _C_PACK, 128)
    out = _stage_c(feats2, W1, W2)
    p = out.reshape(B_PTS // _C_PACK, _C_PACK, 3)
    return jnp.stack(
        [p[:, :, 0].reshape(B_PTS), p[:, :, 1].reshape(B_PTS),
         p[:, :, 2].reshape(B_PTS)], axis=1)


# speedup vs baseline: 1.0584x; 1.0019x over previous
import jax
import jax.numpy as jnp
from jax import lax
from jax.experimental import pallas as pl
from jax.experimental.pallas import tpu as pltpu
from jax.experimental.pallas import tpu_sc as plsc

GRID_H = 256
GRID_W = 256
N_CODES = 1024
F_DIM = 8
B_PTS = 65536

NUM_SC_CORES = 2
NUM_SUBCORES = 16
LANES = 16
NW = NUM_SC_CORES * NUM_SUBCORES
BPW = B_PTS // NW
GROUPS = BPW // LANES


_A_ROWS = 4096


def _argmax_body(hm_ref, out_ref):
    v = hm_ref[...]
    m = jnp.max(v, axis=1, keepdims=True)
    ii = lax.broadcasted_iota(jnp.int32, v.shape, 1)
    sel = jnp.where(v == m, ii, N_CODES)
    idx = jnp.min(sel, axis=1) << 3
    out_ref[...] = idx.reshape(_A_ROWS // 128, 128)


def _stage_a(hm2):
    n_rows = hm2.shape[0]
    return pl.pallas_call(
        _argmax_body,
        grid=(n_rows // _A_ROWS,),
        in_specs=[pl.BlockSpec((_A_ROWS, N_CODES), lambda i: (i, 0))],
        out_specs=pl.BlockSpec((_A_ROWS // 128, 128), lambda i: (i, 0)),
        out_shape=jax.ShapeDtypeStruct((n_rows // 128, 128), jnp.int32),
    )(hm2)


def _sc_body(xq_hbm, yq_hbm, idx_hbm, hf_hbm, out_hbm,
             idxmap_v, hf_v, xq_v, yq_v, feats_v, idx_sh):
    c = lax.axis_index("c")
    s = lax.axis_index("s")
    wid = s * NUM_SC_CORES + c
    base = wid * BPW
    @pl.when(s == 0)
    def _stage_idx():
        pltpu.sync_copy(idx_hbm, idx_sh)

    pltpu.sync_copy(hf_hbm, hf_v)
    pltpu.sync_copy(xq_hbm.at[pl.ds(base, BPW)], xq_v)
    pltpu.sync_copy(yq_hbm.at[pl.ds(base, BPW)], yq_v)
    plsc.subcore_barrier()
    pltpu.sync_copy(idx_sh, idxmap_v)

    lane = lax.iota(jnp.int32, LANES)

    def group(g, carry):
      for sub in range(2):
        q0 = g * (2 * LANES) + sub * LANES
        xv = xq_v[pl.ds(q0, LANES)]
        yv = yq_v[pl.ds(q0, LANES)]
        xs = xv * float(GRID_H)
        ys = yv * float(GRID_W)
        xi = xs.astype(jnp.int32)
        yi = ys.astype(jnp.int32)
        wx = xs - xi.astype(jnp.float32)
        wy = ys - yi.astype(jnp.float32)
        xi1 = jnp.minimum(xi + 1, GRID_H - 1)
        yi1 = jnp.minimum(yi + 1, GRID_W - 1)
        r0 = xi << 8
        r1 = xi1 << 8
        c00 = plsc.load_gather(idxmap_v, [r0 + yi])
        c01 = plsc.load_gather(idxmap_v, [r0 + yi1])
        c10 = plsc.load_gather(idxmap_v, [r1 + yi])
        c11 = plsc.load_gather(idxmap_v, [r1 + yi1])
        omx = 1.0 - wx
        omy = 1.0 - wy
        w00 = omx * omy
        w01 = omx * wy
        w10 = wx * omy
        w11 = wx * wy
        qloc8 = (q0 + lane) << 3
        for f in range(F_DIM):
            a00 = plsc.load_gather(hf_v, [c00 + f])
            a01 = plsc.load_gather(hf_v, [c01 + f])
            a10 = plsc.load_gather(hf_v, [c10 + f])
            a11 = plsc.load_gather(hf_v, [c11 + f])
            acc = w00 * a00 + w01 * a01 + w10 * a10 + w11 * a11
            plsc.store_scatter(feats_v, [qloc8 + f], acc)
      return carry

    lax.fori_loop(0, GROUPS // 2, group, 0)
    pltpu.sync_copy(feats_v, out_hbm.at[pl.ds(base * F_DIM, BPW * F_DIM)])


_SC_CALL_CACHE = []


def _sc_call(*args):
    if not _SC_CALL_CACHE:
        _SC_CALL_CACHE.append(pl.kernel(
            _sc_body,
            out_type=jax.ShapeDtypeStruct((B_PTS * F_DIM,), jnp.float32),
            mesh=plsc.VectorSubcoreMesh(
                core_axis_name="c", subcore_axis_name="s",
                num_cores=NUM_SC_CORES, num_subcores=NUM_SUBCORES),
            compiler_params=pltpu.CompilerParams(needs_layout_passes=False),
            scratch_types=[
                pltpu.VMEM((GRID_H * GRID_W,), jnp.int32),
                pltpu.VMEM((N_CODES * F_DIM,), jnp.float32),
                pltpu.VMEM((BPW,), jnp.float32),
                pltpu.VMEM((BPW,), jnp.float32),
                pltpu.VMEM((BPW * F_DIM,), jnp.float32),
                pltpu.VMEM_SHARED((GRID_H * GRID_W,), jnp.int32),
            ],
        ))
    return _SC_CALL_CACHE[0](*args)


_C_PACK = 128 // F_DIM
_C_ROWS = 2048


def _mlp_body(f_ref, w1_ref, w2_ref, o_ref, w1b_ref, w2b_ref):
    @pl.when(pl.program_id(0) == 0)
    def _build():
        w1b_ref[...] = jnp.zeros((128, 32 * _C_PACK), jnp.float32)
        w2b_ref[...] = jnp.zeros((32 * _C_PACK, 3 * _C_PACK), jnp.float32)
        w1 = w1_ref[...]
        w2 = w2_ref[...]
        for q in range(_C_PACK):
            w1b_ref[q * F_DIM:(q + 1) * F_DIM, q * 32:(q + 1) * 32] = w1
            w2b_ref[q * 32:(q + 1) * 32, q * 3:(q + 1) * 3] = w2

    f = f_ref[...]
    h = jnp.maximum(
        lax.dot(f, w1b_ref[...], preferred_element_type=jnp.float32), 0.0)
    z = lax.dot(h, w2b_ref[...], preferred_element_type=jnp.float32)
    o_ref[...] = 1.0 / (1.0 + jnp.exp(-z))


def _stage_c(feats2, W1, W2):
    n_rows = B_PTS // _C_PACK
    return pl.pallas_call(
        _mlp_body,
        grid=(n_rows // _C_ROWS,),
        in_specs=[
            pl.BlockSpec((_C_ROWS, 128), lambda i: (i, 0)),
            pl.BlockSpec((F_DIM, 32), lambda i: (0, 0)),
            pl.BlockSpec((32, 3), lambda i: (0, 0)),
        ],
        out_specs=pl.BlockSpec((_C_ROWS, 3 * _C_PACK), lambda i: (i, 0)),
        out_shape=jax.ShapeDtypeStruct((n_rows, 3 * _C_PACK), jnp.float32),
        scratch_shapes=[
            pltpu.VMEM((128, 32 * _C_PACK), jnp.float32),
            pltpu.VMEM((32 * _C_PACK, 3 * _C_PACK), jnp.float32),
        ],
    )(feats2, W1, W2)


def kernel(x, hashmap, hash_features, W1, W2):
    hm2 = hashmap.reshape(GRID_H * GRID_W, N_CODES)
    idx_map = _stage_a(hm2).reshape(GRID_H * GRID_W)
    xq = x[:, 0]
    yq = x[:, 1]
    hf_flat = hash_features.reshape(N_CODES * F_DIM)
    feats2 = _sc_call(xq, yq, idx_map, hf_flat).reshape(B_PTS // _C_PACK, 128)
    out = _stage_c(feats2, W1, W2)
    p = out.reshape(B_PTS // _C_PACK, _C_PACK, 3)
    return jnp.stack(
        [p[:, :, 0].reshape(B_PTS), p[:, :, 1].reshape(B_PTS),
         p[:, :, 2].reshape(B_PTS)], axis=1)


# speedup vs baseline: 1.0600x; 1.0015x over previous
import jax
import jax.numpy as jnp
from jax import lax
from jax.experimental import pallas as pl
from jax.experimental.pallas import tpu as pltpu
from jax.experimental.pallas import tpu_sc as plsc

GRID_H = 256
GRID_W = 256
N_CODES = 1024
F_DIM = 8
B_PTS = 65536

NUM_SC_CORES = 2
NUM_SUBCORES = 16
LANES = 16
NW = NUM_SC_CORES * NUM_SUBCORES
BPW = B_PTS // NW
GROUPS = BPW // LANES


_A_ROWS = 4096


def _argmax_body(hm_ref, out_ref):
    v = hm_ref[...]
    m = jnp.max(v, axis=1, keepdims=True)
    ii = lax.broadcasted_iota(jnp.int32, v.shape, 1)
    sel = jnp.where(v == m, ii, N_CODES)
    idx = jnp.min(sel, axis=1) << 3
    out_ref[...] = idx.reshape(_A_ROWS // 128, 128)


def _stage_a(hm2):
    n_rows = hm2.shape[0]
    return pl.pallas_call(
        _argmax_body,
        grid=(n_rows // _A_ROWS,),
        in_specs=[pl.BlockSpec((_A_ROWS, N_CODES), lambda i: (i, 0))],
        out_specs=pl.BlockSpec((_A_ROWS // 128, 128), lambda i: (i, 0)),
        out_shape=jax.ShapeDtypeStruct((n_rows // 128, 128), jnp.int32),
    )(hm2)


def _sc_body(xq_hbm, yq_hbm, idx_hbm, hf_hbm, out_hbm,
             idxmap_v, hf_v, xq_v, yq_v, feats_v, idx_sh):
    c = lax.axis_index("c")
    s = lax.axis_index("s")
    wid = s * NUM_SC_CORES + c
    base = wid * BPW
    @pl.when(s == 0)
    def _stage_idx():
        pltpu.sync_copy(idx_hbm, idx_sh)

    pltpu.sync_copy(hf_hbm, hf_v)
    pltpu.sync_copy(xq_hbm.at[pl.ds(base, BPW)], xq_v)
    pltpu.sync_copy(yq_hbm.at[pl.ds(base, BPW)], yq_v)
    plsc.subcore_barrier()
    pltpu.sync_copy(idx_sh, idxmap_v)

    lane = lax.iota(jnp.int32, LANES)

    def group(g, carry):
        q0 = g * LANES
        xv = xq_v[pl.ds(q0, LANES)]
        yv = yq_v[pl.ds(q0, LANES)]
        xs = xv * float(GRID_H)
        ys = yv * float(GRID_W)
        xi = xs.astype(jnp.int32)
        yi = ys.astype(jnp.int32)
        wx = xs - xi.astype(jnp.float32)
        wy = ys - yi.astype(jnp.float32)
        xi1 = jnp.minimum(xi + 1, GRID_H - 1)
        yi1 = jnp.minimum(yi + 1, GRID_W - 1)
        r0 = xi << 8
        r1 = xi1 << 8
        c00 = plsc.load_gather(idxmap_v, [r0 + yi])
        c01 = plsc.load_gather(idxmap_v, [r0 + yi1])
        c10 = plsc.load_gather(idxmap_v, [r1 + yi])
        c11 = plsc.load_gather(idxmap_v, [r1 + yi1])
        omx = 1.0 - wx
        omy = 1.0 - wy
        w00 = omx * omy
        w01 = omx * wy
        w10 = wx * omy
        w11 = wx * wy
        qloc8 = (q0 + lane) << 3
        for f in range(F_DIM):
            a00 = plsc.load_gather(hf_v, [c00 + f])
            a01 = plsc.load_gather(hf_v, [c01 + f])
            a10 = plsc.load_gather(hf_v, [c10 + f])
            a11 = plsc.load_gather(hf_v, [c11 + f])
            acc = w00 * a00 + w01 * a01 + w10 * a10 + w11 * a11
            plsc.store_scatter(feats_v, [qloc8 + f], acc)
        return carry

    lax.fori_loop(0, GROUPS, group, 0)
    pltpu.sync_copy(feats_v, out_hbm.at[pl.ds(base * F_DIM, BPW * F_DIM)])


_SC_CALL_CACHE = []


def _sc_call(*args):
    if not _SC_CALL_CACHE:
        _SC_CALL_CACHE.append(pl.kernel(
            _sc_body,
            out_type=jax.ShapeDtypeStruct((B_PTS * F_DIM,), jnp.float32),
            mesh=plsc.VectorSubcoreMesh(
                core_axis_name="c", subcore_axis_name="s",
                num_cores=NUM_SC_CORES, num_subcores=NUM_SUBCORES),
            compiler_params=pltpu.CompilerParams(needs_layout_passes=False),
            scratch_types=[
                pltpu.VMEM((GRID_H * GRID_W,), jnp.int32),
                pltpu.VMEM((N_CODES * F_DIM,), jnp.float32),
                pltpu.VMEM((BPW,), jnp.float32),
                pltpu.VMEM((BPW,), jnp.float32),
                pltpu.VMEM((BPW * F_DIM,), jnp.float32),
                pltpu.VMEM_SHARED((GRID_H * GRID_W,), jnp.int32),
            ],
        ))
    return _SC_CALL_CACHE[0](*args)


_C_PACK = 128 // F_DIM
_C_ROWS = 2048


def _mlp_body(f_ref, w1_ref, w2_ref, o_ref, w1b_ref, w2b_ref):
    @pl.when(pl.program_id(0) == 0)
    def _build():
        w1b_ref[...] = jnp.zeros((128, 32 * _C_PACK), jnp.float32)
        w2b_ref[...] = jnp.zeros((32 * _C_PACK, 3 * _C_PACK), jnp.float32)
        w1 = w1_ref[...]
        w2 = w2_ref[...]
        for q in range(_C_PACK):
            w1b_ref[q * F_DIM:(q + 1) * F_DIM, q * 32:(q + 1) * 32] = w1
            w2b_ref[q * 32:(q + 1) * 32, q * 3:(q + 1) * 3] = w2

    f = f_ref[...]
    h = jnp.maximum(
        lax.dot(f, w1b_ref[...], preferred_element_type=jnp.float32), 0.0)
    z = lax.dot(h, w2b_ref[...], preferred_element_type=jnp.float32)
    o_ref[...] = 1.0 / (1.0 + jnp.exp(-z))


def _stage_c(feats2, W1, W2):
    n_rows = B_PTS // _C_PACK
    return pl.pallas_call(
        _mlp_body,
        grid=(n_rows // _C_ROWS,),
        in_specs=[
            pl.BlockSpec((_C_ROWS, 128), lambda i: (i, 0)),
            pl.BlockSpec((F_DIM, 32), lambda i: (0, 0)),
            pl.BlockSpec((32, 3), lambda i: (0, 0)),
        ],
        out_specs=pl.BlockSpec((_C_ROWS, 3 * _C_PACK), lambda i: (i, 0)),
        out_shape=jax.ShapeDtypeStruct((n_rows, 3 * _C_PACK), jnp.float32),
        scratch_shapes=[
            pltpu.VMEM((128, 32 * _C_PACK), jnp.float32),
            pltpu.VMEM((32 * _C_PACK, 3 * _C_PACK), jnp.float32),
        ],
    )(feats2, W1, W2)


def kernel(x, hashmap, hash_features, W1, W2):
    hm2 = hashmap.reshape(GRID_H * GRID_W, N_CODES)
    idx_map = _stage_a(hm2).reshape(GRID_H * GRID_W)
    xq = x[:, 0]
    yq = x[:, 1]
    hf_flat = hash_features.reshape(N_CODES * F_DIM)
    feats2 = _sc_call(xq, yq, idx_map, hf_flat).reshape(B_PTS // _C_PACK, 128)
    out = _stage_c(feats2, W1, W2)
    p = out.reshape(B_PTS // _C_PACK, _C_PACK, 3)
    return jnp.stack(
        [p[:, :, 0].reshape(B_PTS), p[:, :, 1].reshape(B_PTS),
         p[:, :, 2].reshape(B_PTS)], axis=1)
